# baseline (speedup 1.0000x reference)
import jax
import jax.numpy as jnp
import numpy as np
from jax import lax
from jax.experimental import pallas as pl
from jax.experimental.pallas import tpu as pltpu
from jax.experimental.pallas import tpu_sc as plsc

N = 10000
E = 320000
D = 128
DH = D // 2

NC = 2
NS = 16

ROWS_PER_TILE = 656
NPAD = ROWS_PER_TILE * NS
IDX_ROWS = E // 128
ROWS_PER_T = IDX_ROWS // NS
TAIL_ROWS = IDX_ROWS - ROWS_PER_T * NS
WIN = 4
NWIN = ROWS_PER_T // WIN

_BN_SCALE = float(1.0 / np.sqrt(1.0 + 1e-5))


def _sc_mesh():
    return plsc.VectorSubcoreMesh(
        core_axis_name="c", subcore_axis_name="s", num_cores=NC,
        num_subcores=NS)


_DEG_CH = 6
_DEG_PER_W = IDX_ROWS // (NC * NS)
_DEG_TAIL = IDX_ROWS - _DEG_PER_W * NC * NS


def _deg_body(sd2_hbm, out_hbm, idx2_v, ones_v, zbuf_v, acc_sh):
    cid = lax.axis_index("c")
    sid = lax.axis_index("s")
    wid = cid * NS + sid
    for k in range(8):
        ones_v[pl.ds(k * 16, 16)] = jnp.ones((16,), jnp.float32)

    def zstep(t, carry):
        zbuf_v[pl.ds(t * 16, 16)] = jnp.zeros((16,), jnp.float32)
        return carry

    lax.fori_loop(0, ROWS_PER_TILE // 16, zstep, 0)
    pltpu.sync_copy(zbuf_v,
                    acc_sh.at[pl.ds(sid * ROWS_PER_TILE, ROWS_PER_TILE)])
    plsc.subcore_barrier()

    base = wid * _DEG_PER_W

    def step(t, carry):
        pltpu.sync_copy(sd2_hbm.at[1, pl.ds(base + t * _DEG_CH, _DEG_CH)],
                        idx2_v)
        for j in range(_DEG_CH):
            pltpu.sync_copy(ones_v, acc_sh.at[idx2_v.at[j]], add=True)
        return carry

    lax.fori_loop(0, _DEG_PER_W // _DEG_CH, step, 0)

    @pl.when(wid < _DEG_TAIL)
    def _tail():
        pltpu.sync_copy(
            sd2_hbm.at[1, pl.ds(_DEG_PER_W * NC * NS + wid, 1)],
            idx2_v.at[pl.ds(0, 1)])
        pltpu.sync_copy(ones_v, acc_sh.at[idx2_v.at[0]], add=True)

    plsc.subcore_barrier()
    pltpu.sync_copy(acc_sh.at[pl.ds(sid * ROWS_PER_TILE, ROWS_PER_TILE)],
                    zbuf_v)
    pltpu.sync_copy(
        zbuf_v,
        out_hbm.at[pl.ds(cid * NPAD + sid * ROWS_PER_TILE, ROWS_PER_TILE)])


def _deg_partials(sd2):
    return pl.kernel(
        _deg_body,
        out_type=jax.ShapeDtypeStruct((NC * NPAD,), jnp.float32),
        mesh=_sc_mesh(),
        compiler_params=pltpu.CompilerParams(use_tc_tiling_on_sc=False),
        scratch_types=[
            pltpu.VMEM((_DEG_CH, 128), jnp.int32),
            pltpu.VMEM((128,), jnp.float32),
            pltpu.VMEM((ROWS_PER_TILE,), jnp.float32),
            pltpu.VMEM_SHARED((NPAD,), jnp.float32),
        ],
    )(sd2)


def _scat_body(gh_hbm, sd2_hbm, out_hbm,
               idx_a, rows_a, semg_a,
               idx_b, rows_b, semg_b, acc_sh):
    cid = lax.axis_index("c")
    sid = lax.axis_index("s")
    r0 = sid * ROWS_PER_TILE
    nbuf = WIN * 128

    def zstep(t, carry):
        for k in range(DH // 32):
            rows_a[t, pl.ds(k * 32, 32)] = jnp.zeros((32,), jnp.bfloat16)
        return carry

    lax.fori_loop(0, nbuf, zstep, 0)
    pltpu.sync_copy(rows_a, acc_sh.at[pl.ds(r0, nbuf)])
    rem = ROWS_PER_TILE - nbuf
    pltpu.sync_copy(rows_a.at[pl.ds(0, rem)],
                    acc_sh.at[pl.ds(r0 + nbuf, rem)])
    plsc.subcore_barrier()

    base = sid * ROWS_PER_T

    def fire(t, idx, rows, semg):
        pltpu.sync_copy(sd2_hbm.at[0, pl.ds(base + t * WIN, WIN)],
                        idx.at[0])
        pltpu.sync_copy(sd2_hbm.at[1, pl.ds(base + t * WIN, WIN)],
                        idx.at[1])
        for j in range(WIN):
            pltpu.async_copy(gh_hbm.at[cid].at[idx.at[0, j]],
                             rows.at[pl.ds(j * 128, 128)], semg)

    def drain_scatter(idx, rows, semg):
        pltpu.make_async_copy(gh_hbm.at[cid].at[pl.ds(0, nbuf)], rows,
                              semg).wait()
        for j in range(WIN):
            pltpu.sync_copy(rows.at[pl.ds(j * 128, 128)],
                            acc_sh.at[idx.at[1, j]], add=True)

    fire(0, idx_a, rows_a, semg_a)

    def step(u, carry):
        t = 2 * u
        fire(t + 1, idx_b, rows_b, semg_b)
        drain_scatter(idx_a, rows_a, semg_a)
        fire(t + 2, idx_a, rows_a, semg_a)
        drain_scatter(idx_b, rows_b, semg_b)
        return carry

    lax.fori_loop(0, (NWIN - 1) // 2, step, 0)
    drain_scatter(idx_a, rows_a, semg_a)

    @pl.when(sid < TAIL_ROWS)
    def _tail():
        trow = ROWS_PER_T * NS + sid
        pltpu.sync_copy(sd2_hbm.at[0, pl.ds(trow, 1)],
                        idx_b.at[0, pl.ds(0, 1)])
        pltpu.sync_copy(sd2_hbm.at[1, pl.ds(trow, 1)],
                        idx_b.at[1, pl.ds(0, 1)])
        pltpu.async_copy(gh_hbm.at[cid].at[idx_b.at[0, 0]],
                         rows_b.at[pl.ds(0, 128)], semg_b)
        pltpu.make_async_copy(gh_hbm.at[cid].at[pl.ds(0, 128)],
                              rows_b.at[pl.ds(0, 128)], semg_b).wait()
        pltpu.sync_copy(rows_b.at[pl.ds(0, 128)],
                        acc_sh.at[idx_b.at[1, 0]], add=True)

    plsc.subcore_barrier()
    pltpu.sync_copy(acc_sh.at[pl.ds(r0, nbuf)], rows_a)
    pltpu.sync_copy(rows_a, out_hbm.at[cid, pl.ds(r0, nbuf)])
    pltpu.sync_copy(acc_sh.at[pl.ds(r0 + nbuf, rem)], rows_b.at[pl.ds(0, rem)])
    pltpu.sync_copy(rows_b.at[pl.ds(0, rem)],
                    out_hbm.at[cid, pl.ds(r0 + nbuf, rem)])


def _scatter_partials(gh, sd2):
    return pl.kernel(
        _scat_body,
        out_type=jax.ShapeDtypeStruct((NC, NPAD, DH), jnp.bfloat16),
        mesh=_sc_mesh(),
        compiler_params=pltpu.CompilerParams(use_tc_tiling_on_sc=False),
        scratch_types=[
            pltpu.VMEM((2, WIN, 128), jnp.int32),
            pltpu.VMEM((WIN * 128, DH), jnp.bfloat16),
            pltpu.SemaphoreType.DMA,
            pltpu.VMEM((2, WIN, 128), jnp.int32),
            pltpu.VMEM((WIN * 128, DH), jnp.bfloat16),
            pltpu.SemaphoreType.DMA,
            pltpu.VMEM_SHARED((NPAD, DH), jnp.bfloat16),
        ],
    )(gh, sd2)


_BLK = 1000
_GRID = N // _BLK


def _k1a_body(x_ref, w1_ref, h_ref):
    h_ref[...] = jnp.dot(x_ref[...], w1_ref[...],
                         preferred_element_type=jnp.float32)


def _k1a(x, W1):
    return pl.pallas_call(
        _k1a_body,
        grid=(_GRID,),
        in_specs=[
            pl.BlockSpec((_BLK, D), lambda i: (i, 0)),
            pl.BlockSpec((D, D), lambda i: (0, 0)),
        ],
        out_specs=pl.BlockSpec((_BLK, D), lambda i: (i, 0)),
        out_shape=jax.ShapeDtypeStruct((N, D), jnp.float32),
    )(x, W1)


def _k1b_body(h_ref, ds_ref, gh_ref, dinv_ref):
    deg = ds_ref[...] + 1.0
    dinv = lax.rsqrt(deg)
    g = (h_ref[...] * dinv).astype(jnp.bfloat16)
    gh_ref[0] = g[:, :DH]
    gh_ref[1] = g[:, DH:]
    dinv_ref[...] = dinv


def _k1b(h1, dsum):
    return pl.pallas_call(
        _k1b_body,
        grid=(_GRID,),
        in_specs=[
            pl.BlockSpec((_BLK, D), lambda i: (i, 0)),
            pl.BlockSpec((_BLK, 1), lambda i: (i, 0)),
        ],
        out_specs=[
            pl.BlockSpec((NC, _BLK, DH), lambda i: (0, i, 0)),
            pl.BlockSpec((_BLK, 1), lambda i: (i, 0)),
        ],
        out_shape=[
            jax.ShapeDtypeStruct((NC, N, DH), jnp.bfloat16),
            jax.ShapeDtypeStruct((N, 1), jnp.float32),
        ],
    )(h1, dsum)


def _k2_body(p_ref, g1_ref, dinv_ref, b1_ref, gam_ref, bet_ref, w2_ref,
             gh2_ref):
    s = jnp.concatenate(
        [p_ref[0].astype(jnp.float32) + g1_ref[0].astype(jnp.float32),
         p_ref[1].astype(jnp.float32) + g1_ref[1].astype(jnp.float32)],
        axis=1)
    conv1 = s * dinv_ref[...] + b1_ref[...]
    z = jnp.maximum(conv1 * (gam_ref[...] * _BN_SCALE) + bet_ref[...], 0.0)
    h2 = jnp.dot(z, w2_ref[...], preferred_element_type=jnp.float32)
    g2 = (h2 * dinv_ref[...]).astype(jnp.bfloat16)
    gh2_ref[0] = g2[:, :DH]
    gh2_ref[1] = g2[:, DH:]


def _k2(P1, gh1, dinv, b1, gamma, beta, W2):
    return pl.pallas_call(
        _k2_body,
        grid=(_GRID,),
        in_specs=[
            pl.BlockSpec((NC, _BLK, DH), lambda i: (0, i, 0)),
            pl.BlockSpec((NC, _BLK, DH), lambda i: (0, i, 0)),
            pl.BlockSpec((_BLK, 1), lambda i: (i, 0)),
            pl.BlockSpec((1, D), lambda i: (0, 0)),
            pl.BlockSpec((1, D), lambda i: (0, 0)),
            pl.BlockSpec((1, D), lambda i: (0, 0)),
            pl.BlockSpec((D, D), lambda i: (0, 0)),
        ],
        out_specs=pl.BlockSpec((NC, _BLK, DH), lambda i: (0, i, 0)),
        out_shape=jax.ShapeDtypeStruct((NC, N, DH), jnp.bfloat16),
    )(P1, gh1, dinv, b1, gamma, beta, W2)


def _k3_body(p_ref, g2_ref, dinv_ref, b2_ref, x_ref, out_ref):
    s = jnp.concatenate(
        [p_ref[0].astype(jnp.float32) + g2_ref[0].astype(jnp.float32),
         p_ref[1].astype(jnp.float32) + g2_ref[1].astype(jnp.float32)],
        axis=1)
    out_ref[...] = s * dinv_ref[...] + b2_ref[...] + x_ref[...]


def _k3(P2, gh2, dinv, b2, x):
    return pl.pallas_call(
        _k3_body,
        grid=(_GRID,),
        in_specs=[
            pl.BlockSpec((NC, _BLK, DH), lambda i: (0, i, 0)),
            pl.BlockSpec((NC, _BLK, DH), lambda i: (0, i, 0)),
            pl.BlockSpec((_BLK, 1), lambda i: (i, 0)),
            pl.BlockSpec((1, D), lambda i: (0, 0)),
            pl.BlockSpec((_BLK, D), lambda i: (i, 0)),
        ],
        out_specs=pl.BlockSpec((_BLK, D), lambda i: (i, 0)),
        out_shape=jax.ShapeDtypeStruct((N, D), jnp.float32),
    )(P2, gh2, dinv, b2, x)


@jax.jit
def kernel(x, edge_index, W1, b1, gamma, beta, W2, b2):
    sd2 = edge_index.reshape(2, IDX_ROWS, 128)

    dp = _deg_partials(sd2)
    dsum = (dp[:NPAD] + dp[NPAD:])[:N].reshape(N, 1)

    h1 = _k1a(x, W1)
    gh1, dinv = _k1b(h1, dsum)
    P1 = _scatter_partials(gh1, sd2)
    gh2 = _k2(P1, gh1, dinv, b1.reshape(1, D), gamma.reshape(1, D),
              beta.reshape(1, D), W2)
    P2 = _scatter_partials(gh2, sd2)
    out = _k3(P2, gh2, dinv, b2.reshape(1, D), x)
    return out


# speedup vs baseline: 1.2340x; 1.2340x over previous
import jax
import jax.numpy as jnp
import numpy as np
from jax import lax
from jax.experimental import pallas as pl
from jax.experimental.pallas import tpu as pltpu
from jax.experimental.pallas import tpu_sc as plsc

N = 10000
E = 320000
D = 128
DH = D // 2

NC = 2
NS = 16

ROWS_PER_TILE = 656
NPAD = ROWS_PER_TILE * NS
IDX_ROWS = E // 128
ROWS_PER_T = IDX_ROWS // NS
TAIL_ROWS = IDX_ROWS - ROWS_PER_T * NS
WIN = 4
NWIN = ROWS_PER_T // WIN

_BN_SCALE = float(1.0 / np.sqrt(1.0 + 1e-5))


def _sc_mesh():
    return plsc.VectorSubcoreMesh(
        core_axis_name="c", subcore_axis_name="s", num_cores=NC,
        num_subcores=NS)


_DEG_CH = 6
_DEG_PER_W = IDX_ROWS // (NC * NS)
_DEG_TAIL = IDX_ROWS - _DEG_PER_W * NC * NS


def _deg_body(sd2_hbm, out_hbm, idx2_v, ones_v, zbuf_v, acc_sh):
    cid = lax.axis_index("c")
    sid = lax.axis_index("s")
    wid = cid * NS + sid
    for k in range(8):
        ones_v[pl.ds(k * 16, 16)] = jnp.ones((16,), jnp.float32)

    def zstep(t, carry):
        zbuf_v[pl.ds(t * 16, 16)] = jnp.zeros((16,), jnp.float32)
        return carry

    lax.fori_loop(0, ROWS_PER_TILE // 16, zstep, 0)
    pltpu.sync_copy(zbuf_v,
                    acc_sh.at[pl.ds(sid * ROWS_PER_TILE, ROWS_PER_TILE)])
    plsc.subcore_barrier()

    base = wid * _DEG_PER_W

    def step(t, carry):
        pltpu.sync_copy(sd2_hbm.at[1, pl.ds(base + t * _DEG_CH, _DEG_CH)],
                        idx2_v)
        for j in range(_DEG_CH):
            pltpu.sync_copy(ones_v, acc_sh.at[idx2_v.at[j]], add=True)
        return carry

    lax.fori_loop(0, _DEG_PER_W // _DEG_CH, step, 0)

    @pl.when(wid < _DEG_TAIL)
    def _tail():
        pltpu.sync_copy(
            sd2_hbm.at[1, pl.ds(_DEG_PER_W * NC * NS + wid, 1)],
            idx2_v.at[pl.ds(0, 1)])
        pltpu.sync_copy(ones_v, acc_sh.at[idx2_v.at[0]], add=True)

    plsc.subcore_barrier()
    pltpu.sync_copy(acc_sh.at[pl.ds(sid * ROWS_PER_TILE, ROWS_PER_TILE)],
                    zbuf_v)
    pltpu.sync_copy(
        zbuf_v,
        out_hbm.at[pl.ds(cid * NPAD + sid * ROWS_PER_TILE, ROWS_PER_TILE)])


def _deg_partials(sd2):
    return pl.kernel(
        _deg_body,
        out_type=jax.ShapeDtypeStruct((NC * NPAD,), jnp.float32),
        mesh=_sc_mesh(),
        compiler_params=pltpu.CompilerParams(use_tc_tiling_on_sc=False),
        scratch_types=[
            pltpu.VMEM((_DEG_CH, 128), jnp.int32),
            pltpu.VMEM((128,), jnp.float32),
            pltpu.VMEM((ROWS_PER_TILE,), jnp.float32),
            pltpu.VMEM_SHARED((NPAD,), jnp.float32),
        ],
    )(sd2)


def _scat_body(gh_hbm, sd2_hbm, out_hbm,
               idx_a, rows_a, semg_a, semi_a,
               idx_b, rows_b, semg_b, semi_b, acc_sh):
    cid = lax.axis_index("c")
    sid = lax.axis_index("s")
    r0 = sid * ROWS_PER_TILE
    nbuf = WIN * 128

    def zstep(t, carry):
        for k in range(DH // 32):
            rows_a[t, pl.ds(k * 32, 32)] = jnp.zeros((32,), jnp.bfloat16)
        return carry

    lax.fori_loop(0, nbuf, zstep, 0)
    pltpu.sync_copy(rows_a, acc_sh.at[pl.ds(r0, nbuf)])
    rem = ROWS_PER_TILE - nbuf
    pltpu.sync_copy(rows_a.at[pl.ds(0, rem)],
                    acc_sh.at[pl.ds(r0 + nbuf, rem)])
    plsc.subcore_barrier()

    base = sid * ROWS_PER_T

    def prefetch_idx(t, idx, semi):
        pltpu.async_copy(sd2_hbm.at[0, pl.ds(base + t * WIN, WIN)],
                         idx.at[0], semi)
        pltpu.async_copy(sd2_hbm.at[1, pl.ds(base + t * WIN, WIN)],
                         idx.at[1], semi)

    def fire(t, idx, rows, semg, semi):
        pltpu.make_async_copy(sd2_hbm.at[:, pl.ds(0, WIN)], idx, semi).wait()
        for j in range(WIN):
            pltpu.async_copy(gh_hbm.at[cid].at[idx.at[0, j]],
                             rows.at[pl.ds(j * 128, 128)], semg)

    def drain_g(rows, semg):
        pltpu.make_async_copy(gh_hbm.at[cid].at[pl.ds(0, nbuf)], rows,
                              semg).wait()

    def scat(idx, rows):
        for j in range(WIN):
            pltpu.sync_copy(rows.at[pl.ds(j * 128, 128)],
                            acc_sh.at[idx.at[1, j]], add=True)

    prefetch_idx(0, idx_a, semi_a)
    prefetch_idx(1, idx_b, semi_b)
    fire(0, idx_a, rows_a, semg_a, semi_a)

    def step(u, carry):
        t = 2 * u
        fire(t + 1, idx_b, rows_b, semg_b, semi_b)
        drain_g(rows_a, semg_a)
        prefetch_idx(t + 2, idx_a, semi_a)
        scat(idx_a, rows_a)
        fire(t + 2, idx_a, rows_a, semg_a, semi_a)
        drain_g(rows_b, semg_b)
        prefetch_idx(t + 3, idx_b, semi_b)
        scat(idx_b, rows_b)
        return carry

    lax.fori_loop(0, (NWIN - 3) // 2, step, 0)
    fire(NWIN - 2, idx_b, rows_b, semg_b, semi_b)
    drain_g(rows_a, semg_a)
    prefetch_idx(NWIN - 1, idx_a, semi_a)
    scat(idx_a, rows_a)
    fire(NWIN - 1, idx_a, rows_a, semg_a, semi_a)
    drain_g(rows_b, semg_b)
    scat(idx_b, rows_b)
    drain_g(rows_a, semg_a)
    scat(idx_a, rows_a)

    @pl.when(sid < TAIL_ROWS)
    def _tail():
        trow = ROWS_PER_T * NS + sid
        pltpu.sync_copy(sd2_hbm.at[0, pl.ds(trow, 1)],
                        idx_b.at[0, pl.ds(0, 1)])
        pltpu.sync_copy(sd2_hbm.at[1, pl.ds(trow, 1)],
                        idx_b.at[1, pl.ds(0, 1)])
        pltpu.async_copy(gh_hbm.at[cid].at[idx_b.at[0, 0]],
                         rows_b.at[pl.ds(0, 128)], semg_b)
        pltpu.make_async_copy(gh_hbm.at[cid].at[pl.ds(0, 128)],
                              rows_b.at[pl.ds(0, 128)], semg_b).wait()
        pltpu.sync_copy(rows_b.at[pl.ds(0, 128)],
                        acc_sh.at[idx_b.at[1, 0]], add=True)

    plsc.subcore_barrier()
    pltpu.sync_copy(acc_sh.at[pl.ds(r0, nbuf)], rows_a)
    pltpu.sync_copy(rows_a, out_hbm.at[cid, pl.ds(r0, nbuf)])
    pltpu.sync_copy(acc_sh.at[pl.ds(r0 + nbuf, rem)], rows_b.at[pl.ds(0, rem)])
    pltpu.sync_copy(rows_b.at[pl.ds(0, rem)],
                    out_hbm.at[cid, pl.ds(r0 + nbuf, rem)])


def _scatter_partials(gh, sd2):
    return pl.kernel(
        _scat_body,
        out_type=jax.ShapeDtypeStruct((NC, NPAD, DH), jnp.bfloat16),
        mesh=_sc_mesh(),
        compiler_params=pltpu.CompilerParams(use_tc_tiling_on_sc=False),
        scratch_types=[
            pltpu.VMEM((2, WIN, 128), jnp.int32),
            pltpu.VMEM((WIN * 128, DH), jnp.bfloat16),
            pltpu.SemaphoreType.DMA,
            pltpu.SemaphoreType.DMA,
            pltpu.VMEM((2, WIN, 128), jnp.int32),
            pltpu.VMEM((WIN * 128, DH), jnp.bfloat16),
            pltpu.SemaphoreType.DMA,
            pltpu.SemaphoreType.DMA,
            pltpu.VMEM_SHARED((NPAD, DH), jnp.bfloat16),
        ],
    )(gh, sd2)


_BLK = 1000
_GRID = N // _BLK


def _k1a_body(x_ref, w1_ref, h_ref):
    h_ref[...] = jnp.dot(x_ref[...], w1_ref[...],
                         preferred_element_type=jnp.float32)


def _k1a(x, W1):
    return pl.pallas_call(
        _k1a_body,
        grid=(_GRID,),
        in_specs=[
            pl.BlockSpec((_BLK, D), lambda i: (i, 0)),
            pl.BlockSpec((D, D), lambda i: (0, 0)),
        ],
        out_specs=pl.BlockSpec((_BLK, D), lambda i: (i, 0)),
        out_shape=jax.ShapeDtypeStruct((N, D), jnp.float32),
    )(x, W1)


def _k1b_body(h_ref, ds_ref, gh_ref, dinv_ref):
    deg = ds_ref[...] + 1.0
    dinv = lax.rsqrt(deg)
    g = (h_ref[...] * dinv).astype(jnp.bfloat16)
    gh_ref[0] = g[:, :DH]
    gh_ref[1] = g[:, DH:]
    dinv_ref[...] = dinv


def _k1b(h1, dsum):
    return pl.pallas_call(
        _k1b_body,
        grid=(_GRID,),
        in_specs=[
            pl.BlockSpec((_BLK, D), lambda i: (i, 0)),
            pl.BlockSpec((_BLK, 1), lambda i: (i, 0)),
        ],
        out_specs=[
            pl.BlockSpec((NC, _BLK, DH), lambda i: (0, i, 0)),
            pl.BlockSpec((_BLK, 1), lambda i: (i, 0)),
        ],
        out_shape=[
            jax.ShapeDtypeStruct((NC, N, DH), jnp.bfloat16),
            jax.ShapeDtypeStruct((N, 1), jnp.float32),
        ],
    )(h1, dsum)


def _k2_body(p_ref, g1_ref, dinv_ref, b1_ref, gam_ref, bet_ref, w2_ref,
             gh2_ref):
    s = jnp.concatenate(
        [p_ref[0].astype(jnp.float32) + g1_ref[0].astype(jnp.float32),
         p_ref[1].astype(jnp.float32) + g1_ref[1].astype(jnp.float32)],
        axis=1)
    conv1 = s * dinv_ref[...] + b1_ref[...]
    z = jnp.maximum(conv1 * (gam_ref[...] * _BN_SCALE) + bet_ref[...], 0.0)
    h2 = jnp.dot(z, w2_ref[...], preferred_element_type=jnp.float32)
    g2 = (h2 * dinv_ref[...]).astype(jnp.bfloat16)
    gh2_ref[0] = g2[:, :DH]
    gh2_ref[1] = g2[:, DH:]


def _k2(P1, gh1, dinv, b1, gamma, beta, W2):
    return pl.pallas_call(
        _k2_body,
        grid=(_GRID,),
        in_specs=[
            pl.BlockSpec((NC, _BLK, DH), lambda i: (0, i, 0)),
            pl.BlockSpec((NC, _BLK, DH), lambda i: (0, i, 0)),
            pl.BlockSpec((_BLK, 1), lambda i: (i, 0)),
            pl.BlockSpec((1, D), lambda i: (0, 0)),
            pl.BlockSpec((1, D), lambda i: (0, 0)),
            pl.BlockSpec((1, D), lambda i: (0, 0)),
            pl.BlockSpec((D, D), lambda i: (0, 0)),
        ],
        out_specs=pl.BlockSpec((NC, _BLK, DH), lambda i: (0, i, 0)),
        out_shape=jax.ShapeDtypeStruct((NC, N, DH), jnp.bfloat16),
    )(P1, gh1, dinv, b1, gamma, beta, W2)


def _k3_body(p_ref, g2_ref, dinv_ref, b2_ref, x_ref, out_ref):
    s = jnp.concatenate(
        [p_ref[0].astype(jnp.float32) + g2_ref[0].astype(jnp.float32),
         p_ref[1].astype(jnp.float32) + g2_ref[1].astype(jnp.float32)],
        axis=1)
    out_ref[...] = s * dinv_ref[...] + b2_ref[...] + x_ref[...]


def _k3(P2, gh2, dinv, b2, x):
    return pl.pallas_call(
        _k3_body,
        grid=(_GRID,),
        in_specs=[
            pl.BlockSpec((NC, _BLK, DH), lambda i: (0, i, 0)),
            pl.BlockSpec((NC, _BLK, DH), lambda i: (0, i, 0)),
            pl.BlockSpec((_BLK, 1), lambda i: (i, 0)),
            pl.BlockSpec((1, D), lambda i: (0, 0)),
            pl.BlockSpec((_BLK, D), lambda i: (i, 0)),
        ],
        out_specs=pl.BlockSpec((_BLK, D), lambda i: (i, 0)),
        out_shape=jax.ShapeDtypeStruct((N, D), jnp.float32),
    )(P2, gh2, dinv, b2, x)


@jax.jit
def kernel(x, edge_index, W1, b1, gamma, beta, W2, b2):
    sd2 = edge_index.reshape(2, IDX_ROWS, 128)

    dp = _deg_partials(sd2)
    dsum = (dp[:NPAD] + dp[NPAD:])[:N].reshape(N, 1)

    h1 = _k1a(x, W1)
    gh1, dinv = _k1b(h1, dsum)
    P1 = _scatter_partials(gh1, sd2)
    gh2 = _k2(P1, gh1, dinv, b1.reshape(1, D), gamma.reshape(1, D),
              beta.reshape(1, D), W2)
    P2 = _scatter_partials(gh2, sd2)
    out = _k3(P2, gh2, dinv, b2.reshape(1, D), x)
    return out


# speedup vs baseline: 1.2549x; 1.0170x over previous
import jax
import jax.numpy as jnp
import numpy as np
from jax import lax
from jax.experimental import pallas as pl
from jax.experimental.pallas import tpu as pltpu
from jax.experimental.pallas import tpu_sc as plsc

N = 10000
E = 320000
D = 128
DH = D // 2

NC = 2
NS = 16

ROWS_PER_TILE = 656
NPAD = ROWS_PER_TILE * NS
IDX_ROWS = E // 128
ROWS_PER_T = IDX_ROWS // NS
TAIL_ROWS = IDX_ROWS - ROWS_PER_T * NS
WIN = 4
NWIN = ROWS_PER_T // WIN

_BN_SCALE = float(1.0 / np.sqrt(1.0 + 1e-5))


def _sc_mesh():
    return plsc.VectorSubcoreMesh(
        core_axis_name="c", subcore_axis_name="s", num_cores=NC,
        num_subcores=NS)


_DEG_CH = 6
_DEG_PER_W = IDX_ROWS // (NC * NS)
_DEG_TAIL = IDX_ROWS - _DEG_PER_W * NC * NS


def _deg_body(sd2_hbm, out_hbm, idx2_v, ones_v, zbuf_v, acc_sh):
    cid = lax.axis_index("c")
    sid = lax.axis_index("s")
    wid = cid * NS + sid
    for k in range(8):
        ones_v[pl.ds(k * 16, 16)] = jnp.ones((16,), jnp.float32)

    def zstep(t, carry):
        zbuf_v[pl.ds(t * 16, 16)] = jnp.zeros((16,), jnp.float32)
        return carry

    lax.fori_loop(0, ROWS_PER_TILE // 16, zstep, 0)
    pltpu.sync_copy(zbuf_v,
                    acc_sh.at[pl.ds(sid * ROWS_PER_TILE, ROWS_PER_TILE)])
    plsc.subcore_barrier()

    base = wid * _DEG_PER_W

    def step(t, carry):
        pltpu.sync_copy(sd2_hbm.at[1, pl.ds(base + t * _DEG_CH, _DEG_CH)],
                        idx2_v)
        for j in range(_DEG_CH):
            pltpu.sync_copy(ones_v, acc_sh.at[idx2_v.at[j]], add=True)
        return carry

    lax.fori_loop(0, _DEG_PER_W // _DEG_CH, step, 0)

    @pl.when(wid < _DEG_TAIL)
    def _tail():
        pltpu.sync_copy(
            sd2_hbm.at[1, pl.ds(_DEG_PER_W * NC * NS + wid, 1)],
            idx2_v.at[pl.ds(0, 1)])
        pltpu.sync_copy(ones_v, acc_sh.at[idx2_v.at[0]], add=True)

    plsc.subcore_barrier()
    pltpu.sync_copy(acc_sh.at[pl.ds(sid * ROWS_PER_TILE, ROWS_PER_TILE)],
                    zbuf_v)
    pltpu.sync_copy(
        zbuf_v,
        out_hbm.at[pl.ds(cid * NPAD + sid * ROWS_PER_TILE, ROWS_PER_TILE)])


def _deg_partials(sd2):
    return pl.kernel(
        _deg_body,
        out_type=jax.ShapeDtypeStruct((NC * NPAD,), jnp.float32),
        mesh=_sc_mesh(),
        compiler_params=pltpu.CompilerParams(use_tc_tiling_on_sc=False),
        scratch_types=[
            pltpu.VMEM((_DEG_CH, 128), jnp.int32),
            pltpu.VMEM((128,), jnp.float32),
            pltpu.VMEM((ROWS_PER_TILE,), jnp.float32),
            pltpu.VMEM_SHARED((NPAD,), jnp.float32),
        ],
    )(sd2)


def _scat_body(gh_hbm, sd2_hbm, out_hbm,
               idx_all, semi, rows_a, semg_a, rows_b, semg_b, acc_sh):
    cid = lax.axis_index("c")
    sid = lax.axis_index("s")
    r0 = sid * ROWS_PER_TILE
    nbuf = WIN * 128
    base = sid * ROWS_PER_T

    pltpu.async_copy(sd2_hbm.at[0, pl.ds(base, ROWS_PER_T)], idx_all.at[0],
                     semi)
    pltpu.async_copy(sd2_hbm.at[1, pl.ds(base, ROWS_PER_T)], idx_all.at[1],
                     semi)

    def zstep(t, carry):
        for k in range(DH // 32):
            rows_a[t, pl.ds(k * 32, 32)] = jnp.zeros((32,), jnp.bfloat16)
        return carry

    lax.fori_loop(0, nbuf, zstep, 0)
    pltpu.sync_copy(rows_a, acc_sh.at[pl.ds(r0, nbuf)])
    rem = ROWS_PER_TILE - nbuf
    pltpu.sync_copy(rows_a.at[pl.ds(0, rem)],
                    acc_sh.at[pl.ds(r0 + nbuf, rem)])
    pltpu.make_async_copy(sd2_hbm.at[:, pl.ds(0, ROWS_PER_T)], idx_all,
                          semi).wait()
    plsc.subcore_barrier()

    def fire(t, rows, semg):
        for j in range(WIN):
            pltpu.async_copy(gh_hbm.at[cid].at[idx_all.at[0, t * WIN + j]],
                             rows.at[pl.ds(j * 128, 128)], semg)

    def drain_scatter(t, rows, semg):
        pltpu.make_async_copy(gh_hbm.at[cid].at[pl.ds(0, nbuf)], rows,
                              semg).wait()
        for j in range(WIN):
            pltpu.sync_copy(rows.at[pl.ds(j * 128, 128)],
                            acc_sh.at[idx_all.at[1, t * WIN + j]], add=True)

    fire(0, rows_a, semg_a)

    def step(u, carry):
        t = 2 * u
        fire(t + 1, rows_b, semg_b)
        drain_scatter(t, rows_a, semg_a)
        fire(t + 2, rows_a, semg_a)
        drain_scatter(t + 1, rows_b, semg_b)
        return carry

    lax.fori_loop(0, (NWIN - 3) // 2, step, 0)
    fire(NWIN - 2, rows_b, semg_b)
    drain_scatter(NWIN - 3, rows_a, semg_a)
    fire(NWIN - 1, rows_a, semg_a)
    drain_scatter(NWIN - 2, rows_b, semg_b)
    drain_scatter(NWIN - 1, rows_a, semg_a)

    @pl.when(sid < TAIL_ROWS)
    def _tail():
        trow = ROWS_PER_T * NS + sid
        pltpu.sync_copy(sd2_hbm.at[0, pl.ds(trow, 1)],
                        idx_all.at[0, pl.ds(0, 1)])
        pltpu.sync_copy(sd2_hbm.at[1, pl.ds(trow, 1)],
                        idx_all.at[1, pl.ds(0, 1)])
        pltpu.async_copy(gh_hbm.at[cid].at[idx_all.at[0, 0]],
                         rows_b.at[pl.ds(0, 128)], semg_b)
        pltpu.make_async_copy(gh_hbm.at[cid].at[pl.ds(0, 128)],
                              rows_b.at[pl.ds(0, 128)], semg_b).wait()
        pltpu.sync_copy(rows_b.at[pl.ds(0, 128)],
                        acc_sh.at[idx_all.at[1, 0]], add=True)

    plsc.subcore_barrier()
    pltpu.sync_copy(acc_sh.at[pl.ds(r0, nbuf)], rows_a)
    pltpu.sync_copy(rows_a, out_hbm.at[cid, pl.ds(r0, nbuf)])
    pltpu.sync_copy(acc_sh.at[pl.ds(r0 + nbuf, rem)], rows_b.at[pl.ds(0, rem)])
    pltpu.sync_copy(rows_b.at[pl.ds(0, rem)],
                    out_hbm.at[cid, pl.ds(r0 + nbuf, rem)])


def _scatter_partials(gh, sd2):
    return pl.kernel(
        _scat_body,
        out_type=jax.ShapeDtypeStruct((NC, NPAD, DH), jnp.bfloat16),
        mesh=_sc_mesh(),
        compiler_params=pltpu.CompilerParams(use_tc_tiling_on_sc=False),
        scratch_types=[
            pltpu.VMEM((2, ROWS_PER_T, 128), jnp.int32),
            pltpu.SemaphoreType.DMA,
            pltpu.VMEM((WIN * 128, DH), jnp.bfloat16),
            pltpu.SemaphoreType.DMA,
            pltpu.VMEM((WIN * 128, DH), jnp.bfloat16),
            pltpu.SemaphoreType.DMA,
            pltpu.VMEM_SHARED((NPAD, DH), jnp.bfloat16),
        ],
    )(gh, sd2)


_BLK = 1000
_GRID = N // _BLK


def _k1a_body(x_ref, w1_ref, h_ref):
    h_ref[...] = jnp.dot(x_ref[...], w1_ref[...],
                         preferred_element_type=jnp.float32)


def _k1a(x, W1):
    return pl.pallas_call(
        _k1a_body,
        grid=(_GRID,),
        in_specs=[
            pl.BlockSpec((_BLK, D), lambda i: (i, 0)),
            pl.BlockSpec((D, D), lambda i: (0, 0)),
        ],
        out_specs=pl.BlockSpec((_BLK, D), lambda i: (i, 0)),
        out_shape=jax.ShapeDtypeStruct((N, D), jnp.float32),
    )(x, W1)


def _k1b_body(h_ref, ds_ref, gh_ref, dinv_ref):
    deg = ds_ref[...] + 1.0
    dinv = lax.rsqrt(deg)
    g = (h_ref[...] * dinv).astype(jnp.bfloat16)
    gh_ref[0] = g[:, :DH]
    gh_ref[1] = g[:, DH:]
    dinv_ref[...] = dinv


def _k1b(h1, dsum):
    return pl.pallas_call(
        _k1b_body,
        grid=(_GRID,),
        in_specs=[
            pl.BlockSpec((_BLK, D), lambda i: (i, 0)),
            pl.BlockSpec((_BLK, 1), lambda i: (i, 0)),
        ],
        out_specs=[
            pl.BlockSpec((NC, _BLK, DH), lambda i: (0, i, 0)),
            pl.BlockSpec((_BLK, 1), lambda i: (i, 0)),
        ],
        out_shape=[
            jax.ShapeDtypeStruct((NC, N, DH), jnp.bfloat16),
            jax.ShapeDtypeStruct((N, 1), jnp.float32),
        ],
    )(h1, dsum)


def _k2_body(p_ref, g1_ref, dinv_ref, b1_ref, gam_ref, bet_ref, w2_ref,
             gh2_ref):
    s = jnp.concatenate(
        [p_ref[0].astype(jnp.float32) + g1_ref[0].astype(jnp.float32),
         p_ref[1].astype(jnp.float32) + g1_ref[1].astype(jnp.float32)],
        axis=1)
    conv1 = s * dinv_ref[...] + b1_ref[...]
    z = jnp.maximum(conv1 * (gam_ref[...] * _BN_SCALE) + bet_ref[...], 0.0)
    h2 = jnp.dot(z, w2_ref[...], preferred_element_type=jnp.float32)
    g2 = (h2 * dinv_ref[...]).astype(jnp.bfloat16)
    gh2_ref[0] = g2[:, :DH]
    gh2_ref[1] = g2[:, DH:]


def _k2(P1, gh1, dinv, b1, gamma, beta, W2):
    return pl.pallas_call(
        _k2_body,
        grid=(_GRID,),
        in_specs=[
            pl.BlockSpec((NC, _BLK, DH), lambda i: (0, i, 0)),
            pl.BlockSpec((NC, _BLK, DH), lambda i: (0, i, 0)),
            pl.BlockSpec((_BLK, 1), lambda i: (i, 0)),
            pl.BlockSpec((1, D), lambda i: (0, 0)),
            pl.BlockSpec((1, D), lambda i: (0, 0)),
            pl.BlockSpec((1, D), lambda i: (0, 0)),
            pl.BlockSpec((D, D), lambda i: (0, 0)),
        ],
        out_specs=pl.BlockSpec((NC, _BLK, DH), lambda i: (0, i, 0)),
        out_shape=jax.ShapeDtypeStruct((NC, N, DH), jnp.bfloat16),
    )(P1, gh1, dinv, b1, gamma, beta, W2)


def _k3_body(p_ref, g2_ref, dinv_ref, b2_ref, x_ref, out_ref):
    s = jnp.concatenate(
        [p_ref[0].astype(jnp.float32) + g2_ref[0].astype(jnp.float32),
         p_ref[1].astype(jnp.float32) + g2_ref[1].astype(jnp.float32)],
        axis=1)
    out_ref[...] = s * dinv_ref[...] + b2_ref[...] + x_ref[...]


def _k3(P2, gh2, dinv, b2, x):
    return pl.pallas_call(
        _k3_body,
        grid=(_GRID,),
        in_specs=[
            pl.BlockSpec((NC, _BLK, DH), lambda i: (0, i, 0)),
            pl.BlockSpec((NC, _BLK, DH), lambda i: (0, i, 0)),
            pl.BlockSpec((_BLK, 1), lambda i: (i, 0)),
            pl.BlockSpec((1, D), lambda i: (0, 0)),
            pl.BlockSpec((_BLK, D), lambda i: (i, 0)),
        ],
        out_specs=pl.BlockSpec((_BLK, D), lambda i: (i, 0)),
        out_shape=jax.ShapeDtypeStruct((N, D), jnp.float32),
    )(P2, gh2, dinv, b2, x)


@jax.jit
def kernel(x, edge_index, W1, b1, gamma, beta, W2, b2):
    sd2 = edge_index.reshape(2, IDX_ROWS, 128)

    dp = _deg_partials(sd2)
    dsum = (dp[:NPAD] + dp[NPAD:])[:N].reshape(N, 1)

    h1 = _k1a(x, W1)
    gh1, dinv = _k1b(h1, dsum)
    P1 = _scatter_partials(gh1, sd2)
    gh2 = _k2(P1, gh1, dinv, b1.reshape(1, D), gamma.reshape(1, D),
              beta.reshape(1, D), W2)
    P2 = _scatter_partials(gh2, sd2)
    out = _k3(P2, gh2, dinv, b2.reshape(1, D), x)
    return out


# speedup vs baseline: 1.2876x; 1.0260x over previous
import jax
import jax.numpy as jnp
import numpy as np
from jax import lax
from jax.experimental import pallas as pl
from jax.experimental.pallas import tpu as pltpu
from jax.experimental.pallas import tpu_sc as plsc

N = 10000
E = 320000
D = 128
DH = D // 2

NC = 2
NS = 16

ROWS_PER_TILE = 656
NPAD = ROWS_PER_TILE * NS
IDX_ROWS = E // 128
ROWS_PER_T = IDX_ROWS // NS
TAIL_ROWS = IDX_ROWS - ROWS_PER_T * NS
WIN = 4
NWIN = ROWS_PER_T // WIN

_BN_SCALE = float(1.0 / np.sqrt(1.0 + 1e-5))


def _sc_mesh():
    return plsc.VectorSubcoreMesh(
        core_axis_name="c", subcore_axis_name="s", num_cores=NC,
        num_subcores=NS)


_DEG_CH = 6
_DEG_PER_W = IDX_ROWS // (NC * NS)
_DEG_TAIL = IDX_ROWS - _DEG_PER_W * NC * NS


def _deg_body(sd2_hbm, out_hbm, idx2_v, semi, ones_v, zbuf_v, acc_sh):
    cid = lax.axis_index("c")
    sid = lax.axis_index("s")
    wid = cid * NS + sid
    base = wid * _DEG_PER_W
    pltpu.async_copy(sd2_hbm.at[1, pl.ds(base, _DEG_PER_W)], idx2_v, semi)
    for k in range(8):
        ones_v[pl.ds(k * 16, 16)] = jnp.ones((16,), jnp.float32)

    def zstep(t, carry):
        zbuf_v[pl.ds(t * 16, 16)] = jnp.zeros((16,), jnp.float32)
        return carry

    lax.fori_loop(0, ROWS_PER_TILE // 16, zstep, 0)
    pltpu.sync_copy(zbuf_v,
                    acc_sh.at[pl.ds(sid * ROWS_PER_TILE, ROWS_PER_TILE)])
    pltpu.make_async_copy(sd2_hbm.at[1, pl.ds(0, _DEG_PER_W)], idx2_v,
                          semi).wait()
    plsc.subcore_barrier()

    def step(j, carry):
        pltpu.sync_copy(ones_v, acc_sh.at[idx2_v.at[j]], add=True)
        return carry

    lax.fori_loop(0, _DEG_PER_W, step, 0)

    @pl.when(wid < _DEG_TAIL)
    def _tail():
        pltpu.sync_copy(
            sd2_hbm.at[1, pl.ds(_DEG_PER_W * NC * NS + wid, 1)],
            idx2_v.at[pl.ds(0, 1)])
        pltpu.sync_copy(ones_v, acc_sh.at[idx2_v.at[0]], add=True)

    plsc.subcore_barrier()
    pltpu.sync_copy(acc_sh.at[pl.ds(sid * ROWS_PER_TILE, ROWS_PER_TILE)],
                    zbuf_v)
    pltpu.sync_copy(
        zbuf_v,
        out_hbm.at[pl.ds(cid * NPAD + sid * ROWS_PER_TILE, ROWS_PER_TILE)])


def _deg_partials(sd2):
    return pl.kernel(
        _deg_body,
        out_type=jax.ShapeDtypeStruct((NC * NPAD,), jnp.float32),
        mesh=_sc_mesh(),
        compiler_params=pltpu.CompilerParams(use_tc_tiling_on_sc=False),
        scratch_types=[
            pltpu.VMEM((_DEG_PER_W, 128), jnp.int32),
            pltpu.SemaphoreType.DMA,
            pltpu.VMEM((128,), jnp.float32),
            pltpu.VMEM((ROWS_PER_TILE,), jnp.float32),
            pltpu.VMEM_SHARED((NPAD,), jnp.float32),
        ],
    )(sd2)


def _scat_body(gh_hbm, sd2_hbm, out_hbm,
               idx_all, semi, rows_a, semg_a, rows_b, semg_b, acc_sh):
    cid = lax.axis_index("c")
    sid = lax.axis_index("s")
    r0 = sid * ROWS_PER_TILE
    nbuf = WIN * 128
    base = sid * ROWS_PER_T

    pltpu.async_copy(sd2_hbm.at[0, pl.ds(base, ROWS_PER_T)], idx_all.at[0],
                     semi)
    pltpu.async_copy(sd2_hbm.at[1, pl.ds(base, ROWS_PER_T)], idx_all.at[1],
                     semi)

    def zstep(t, carry):
        for k in range(DH // 32):
            rows_a[t, pl.ds(k * 32, 32)] = jnp.zeros((32,), jnp.bfloat16)
        return carry

    lax.fori_loop(0, nbuf, zstep, 0)
    pltpu.sync_copy(rows_a, acc_sh.at[pl.ds(r0, nbuf)])
    rem = ROWS_PER_TILE - nbuf
    pltpu.sync_copy(rows_a.at[pl.ds(0, rem)],
                    acc_sh.at[pl.ds(r0 + nbuf, rem)])
    pltpu.make_async_copy(sd2_hbm.at[:, pl.ds(0, ROWS_PER_T)], idx_all,
                          semi).wait()
    plsc.subcore_barrier()

    def fire(t, rows, semg):
        for j in range(WIN):
            pltpu.async_copy(gh_hbm.at[cid].at[idx_all.at[0, t * WIN + j]],
                             rows.at[pl.ds(j * 128, 128)], semg)

    def drain_scatter(t, rows, semg):
        pltpu.make_async_copy(gh_hbm.at[cid].at[pl.ds(0, nbuf)], rows,
                              semg).wait()
        for j in range(WIN):
            pltpu.sync_copy(rows.at[pl.ds(j * 128, 128)],
                            acc_sh.at[idx_all.at[1, t * WIN + j]], add=True)

    fire(0, rows_a, semg_a)

    def step(u, carry):
        t = 2 * u
        fire(t + 1, rows_b, semg_b)
        drain_scatter(t, rows_a, semg_a)
        fire(t + 2, rows_a, semg_a)
        drain_scatter(t + 1, rows_b, semg_b)
        return carry

    lax.fori_loop(0, (NWIN - 3) // 2, step, 0)
    fire(NWIN - 2, rows_b, semg_b)
    drain_scatter(NWIN - 3, rows_a, semg_a)
    fire(NWIN - 1, rows_a, semg_a)
    drain_scatter(NWIN - 2, rows_b, semg_b)
    drain_scatter(NWIN - 1, rows_a, semg_a)

    @pl.when(sid < TAIL_ROWS)
    def _tail():
        trow = ROWS_PER_T * NS + sid
        pltpu.sync_copy(sd2_hbm.at[0, pl.ds(trow, 1)],
                        idx_all.at[0, pl.ds(0, 1)])
        pltpu.sync_copy(sd2_hbm.at[1, pl.ds(trow, 1)],
                        idx_all.at[1, pl.ds(0, 1)])
        pltpu.async_copy(gh_hbm.at[cid].at[idx_all.at[0, 0]],
                         rows_b.at[pl.ds(0, 128)], semg_b)
        pltpu.make_async_copy(gh_hbm.at[cid].at[pl.ds(0, 128)],
                              rows_b.at[pl.ds(0, 128)], semg_b).wait()
        pltpu.sync_copy(rows_b.at[pl.ds(0, 128)],
                        acc_sh.at[idx_all.at[1, 0]], add=True)

    plsc.subcore_barrier()
    pltpu.sync_copy(acc_sh.at[pl.ds(r0, nbuf)], rows_a)
    pltpu.sync_copy(rows_a, out_hbm.at[cid, pl.ds(r0, nbuf)])
    pltpu.sync_copy(acc_sh.at[pl.ds(r0 + nbuf, rem)], rows_b.at[pl.ds(0, rem)])
    pltpu.sync_copy(rows_b.at[pl.ds(0, rem)],
                    out_hbm.at[cid, pl.ds(r0 + nbuf, rem)])


def _scatter_partials(gh, sd2):
    return pl.kernel(
        _scat_body,
        out_type=jax.ShapeDtypeStruct((NC, NPAD, DH), jnp.bfloat16),
        mesh=_sc_mesh(),
        compiler_params=pltpu.CompilerParams(use_tc_tiling_on_sc=False),
        scratch_types=[
            pltpu.VMEM((2, ROWS_PER_T, 128), jnp.int32),
            pltpu.SemaphoreType.DMA,
            pltpu.VMEM((WIN * 128, DH), jnp.bfloat16),
            pltpu.SemaphoreType.DMA,
            pltpu.VMEM((WIN * 128, DH), jnp.bfloat16),
            pltpu.SemaphoreType.DMA,
            pltpu.VMEM_SHARED((NPAD, DH), jnp.bfloat16),
        ],
    )(gh, sd2)


_BLK = 1000
_GRID = N // _BLK


def _k1a_body(x_ref, w1_ref, h_ref):
    h_ref[...] = jnp.dot(x_ref[...], w1_ref[...],
                         preferred_element_type=jnp.float32)


def _k1a(x, W1):
    return pl.pallas_call(
        _k1a_body,
        grid=(_GRID,),
        in_specs=[
            pl.BlockSpec((_BLK, D), lambda i: (i, 0)),
            pl.BlockSpec((D, D), lambda i: (0, 0)),
        ],
        out_specs=pl.BlockSpec((_BLK, D), lambda i: (i, 0)),
        out_shape=jax.ShapeDtypeStruct((N, D), jnp.float32),
    )(x, W1)


def _k1b_body(h_ref, ds_ref, gh_ref, dinv_ref):
    deg = ds_ref[...] + 1.0
    dinv = lax.rsqrt(deg)
    g = (h_ref[...] * dinv).astype(jnp.bfloat16)
    gh_ref[0] = g[:, :DH]
    gh_ref[1] = g[:, DH:]
    dinv_ref[...] = dinv


def _k1b(h1, dsum):
    return pl.pallas_call(
        _k1b_body,
        grid=(_GRID,),
        in_specs=[
            pl.BlockSpec((_BLK, D), lambda i: (i, 0)),
            pl.BlockSpec((_BLK, 1), lambda i: (i, 0)),
        ],
        out_specs=[
            pl.BlockSpec((NC, _BLK, DH), lambda i: (0, i, 0)),
            pl.BlockSpec((_BLK, 1), lambda i: (i, 0)),
        ],
        out_shape=[
            jax.ShapeDtypeStruct((NC, N, DH), jnp.bfloat16),
            jax.ShapeDtypeStruct((N, 1), jnp.float32),
        ],
    )(h1, dsum)


def _k2_body(p_ref, g1_ref, dinv_ref, b1_ref, gam_ref, bet_ref, w2_ref,
             gh2_ref):
    s = jnp.concatenate(
        [p_ref[0].astype(jnp.float32) + g1_ref[0].astype(jnp.float32),
         p_ref[1].astype(jnp.float32) + g1_ref[1].astype(jnp.float32)],
        axis=1)
    conv1 = s * dinv_ref[...] + b1_ref[...]
    z = jnp.maximum(conv1 * (gam_ref[...] * _BN_SCALE) + bet_ref[...], 0.0)
    h2 = jnp.dot(z, w2_ref[...], preferred_element_type=jnp.float32)
    g2 = (h2 * dinv_ref[...]).astype(jnp.bfloat16)
    gh2_ref[0] = g2[:, :DH]
    gh2_ref[1] = g2[:, DH:]


def _k2(P1, gh1, dinv, b1, gamma, beta, W2):
    return pl.pallas_call(
        _k2_body,
        grid=(_GRID,),
        in_specs=[
            pl.BlockSpec((NC, _BLK, DH), lambda i: (0, i, 0)),
            pl.BlockSpec((NC, _BLK, DH), lambda i: (0, i, 0)),
            pl.BlockSpec((_BLK, 1), lambda i: (i, 0)),
            pl.BlockSpec((1, D), lambda i: (0, 0)),
            pl.BlockSpec((1, D), lambda i: (0, 0)),
            pl.BlockSpec((1, D), lambda i: (0, 0)),
            pl.BlockSpec((D, D), lambda i: (0, 0)),
        ],
        out_specs=pl.BlockSpec((NC, _BLK, DH), lambda i: (0, i, 0)),
        out_shape=jax.ShapeDtypeStruct((NC, N, DH), jnp.bfloat16),
    )(P1, gh1, dinv, b1, gamma, beta, W2)


def _k3_body(p_ref, g2_ref, dinv_ref, b2_ref, x_ref, out_ref):
    s = jnp.concatenate(
        [p_ref[0].astype(jnp.float32) + g2_ref[0].astype(jnp.float32),
         p_ref[1].astype(jnp.float32) + g2_ref[1].astype(jnp.float32)],
        axis=1)
    out_ref[...] = s * dinv_ref[...] + b2_ref[...] + x_ref[...]


def _k3(P2, gh2, dinv, b2, x):
    return pl.pallas_call(
        _k3_body,
        grid=(_GRID,),
        in_specs=[
            pl.BlockSpec((NC, _BLK, DH), lambda i: (0, i, 0)),
            pl.BlockSpec((NC, _BLK, DH), lambda i: (0, i, 0)),
            pl.BlockSpec((_BLK, 1), lambda i: (i, 0)),
            pl.BlockSpec((1, D), lambda i: (0, 0)),
            pl.BlockSpec((_BLK, D), lambda i: (i, 0)),
        ],
        out_specs=pl.BlockSpec((_BLK, D), lambda i: (i, 0)),
        out_shape=jax.ShapeDtypeStruct((N, D), jnp.float32),
    )(P2, gh2, dinv, b2, x)


@jax.jit
def kernel(x, edge_index, W1, b1, gamma, beta, W2, b2):
    sd2 = edge_index.reshape(2, IDX_ROWS, 128)

    dp = _deg_partials(sd2)
    dsum = (dp[:NPAD] + dp[NPAD:])[:N].reshape(N, 1)

    h1 = _k1a(x, W1)
    gh1, dinv = _k1b(h1, dsum)
    P1 = _scatter_partials(gh1, sd2)
    gh2 = _k2(P1, gh1, dinv, b1.reshape(1, D), gamma.reshape(1, D),
              beta.reshape(1, D), W2)
    P2 = _scatter_partials(gh2, sd2)
    out = _k3(P2, gh2, dinv, b2.reshape(1, D), x)
    return out
